# baseline (speedup 1.0000x reference)
import functools

import jax
import jax.numpy as jnp
from jax import lax
from jax.experimental import pallas as pl
from jax.experimental.pallas import tpu as pltpu
from jax.experimental.pallas import tpu_sc as plsc

N = 10000
E = 80000
DIN = 128
DOUT = 128
R = 4
NB = 2

NC = 2
NS = 16
L = 16

D = 128
NPAD = 10240
RT = NPAD // NS
DR = NPAD // D
CHUNK = 128
NCH = E // CHUNK
CT = NCH // NS
ET = CT * CHUNK
ZR = 16

_mesh = plsc.VectorSubcoreMesh(
    core_axis_name="c", subcore_axis_name="s", num_cores=NC, num_subcores=NS
)


@functools.partial(
    pl.kernel,
    out_type=(
        jax.ShapeDtypeStruct((R, NPAD, D), jnp.float32),
        jax.ShapeDtypeStruct((R, DR, D), jnp.float32),
    ),
    mesh=_mesh,
    compiler_params=pltpu.CompilerParams(needs_layout_passes=False),
    scratch_types=[
        pltpu.VMEM((2, CHUNK), jnp.int32),
        pltpu.VMEM((2, CHUNK), jnp.int32),
        pltpu.VMEM((CHUNK,), jnp.int32),
        pltpu.VMEM((CHUNK,), jnp.int32),
        pltpu.VMEM((CHUNK, D), jnp.float32),
        pltpu.VMEM((CHUNK, D), jnp.float32),
        pltpu.VMEM((ZR, D), jnp.float32),
        pltpu.VMEM((DR, D), jnp.float32),
        pltpu.VMEM((DR,), jnp.int32),
        pltpu.VMEM_SHARED((NPAD, D), jnp.float32),
        pltpu.VMEM_SHARED((DR, D), jnp.float32),
        pltpu.SemaphoreType.DMA,
        pltpu.SemaphoreType.DMA,
        pltpu.SemaphoreType.DMA,
        pltpu.SemaphoreType.DMA,
        pltpu.SemaphoreType.DMA,
        pltpu.SemaphoreType.DMA,
        pltpu.SemaphoreType.DMA,
    ],
)
def _sc_aggregate(x_hbm, e0_hbm, e1_hbm, e2_hbm, e3_hbm, out_hbm, deg_hbm,
                  idxa_v, idxb_v, dsta_v, dstb_v, rowsa_v, rowsb_v, zero_v,
                  degp_v, idxdr_v, acc_sh, deg_sh, semga, semgb, semsa, semsb,
                  semia, semib, semz):
    cid = lax.axis_index("c")
    sid = lax.axis_index("s")

    @pl.loop(0, ZR * D // L)
    def _fill(i):
        zero_v[i // (D // L), pl.ds((i % (D // L)) * L, L)] = (
            jnp.zeros((L,), jnp.float32))

    @pl.loop(0, DR // L)
    def _fill_idx(g):
        idxdr_v[pl.ds(g * L, L)] = lax.iota(jnp.int32, L) + g * L

    my_rows = sid * RT
    base_e = sid * ET
    myc = CT + jnp.where(sid == 0, 1, 0)

    def chunk_off(c):
        return jnp.where(c < CT, base_e + c * CHUNK, (NCH - 1) * CHUNK)

    def start_gather(idx_v, rows_v, semg):
        h = CHUNK // 2
        pltpu.async_copy(x_hbm.at[idx_v.at[0, pl.ds(0, h)]],
                         rows_v.at[pl.ds(0, h)], semg)
        pltpu.async_copy(x_hbm.at[idx_v.at[0, pl.ds(h, h)]],
                         rows_v.at[pl.ds(h, h)], semg)

    def start_idx_load(e_hbm, off, idx_v, semi):
        pltpu.async_copy(e_hbm.at[:, pl.ds(off, CHUNK)], idx_v, semi)

    def wait_idx(e_hbm, idx_v, semi):
        pltpu.make_async_copy(e_hbm.at[:, pl.ds(0, CHUNK)], idx_v,
                              semi).wait()

    def wait_dma(idx_v, rows_v, sem):
        pltpu.make_async_copy(x_hbm.at[idx_v.at[0]], rows_v, sem).wait()

    def do_degree(dst_v):
        lane = lax.iota(jnp.int32, L)
        one = jnp.ones((L,), jnp.float32)
        for g in range(CHUNK // L):
            dst = dst_v[pl.ds(g * L, L)]
            hi = lax.shift_right_logical(dst, 7)
            lo = lax.bitwise_and(dst, 127)
            for k in range(L):
                plsc.addupdate_scatter(degp_v, [hi, lo], one,
                                       mask=lane == k)

    def do_relation(e_hbm, r):
        start_idx_load(e_hbm, chunk_off(0), idxa_v, semia)

        @pl.when(myc >= 2)
        def _prime_idx1():
            start_idx_load(e_hbm, chunk_off(1), idxb_v, semib)
        wait_idx(e_hbm, idxa_v, semia)
        start_gather(idxa_v, rowsa_v, semga)

        @pl.loop(0, RT // ZR)
        def _clear(z):
            pltpu.async_copy(zero_v, acc_sh.at[pl.ds(my_rows + z * ZR, ZR)],
                             semz)

        @pl.when(sid < DR // 8)
        def _clear_deg():
            pltpu.async_copy(zero_v.at[pl.ds(0, 8)],
                             deg_sh.at[pl.ds(sid * 8, 8)], semz)

        @pl.loop(0, DR * (D // L))
        def _clear_degp(i):
            degp_v[i // (D // L), pl.ds((i % (D // L)) * L, L)] = (
                jnp.zeros((L,), jnp.float32))

        @pl.loop(0, RT // ZR)
        def _drain_clear(z):
            pltpu.make_async_copy(
                zero_v, acc_sh.at[pl.ds(my_rows, ZR)], semz).wait()

        @pl.when(sid < DR // 8)
        def _drain_clear_deg():
            pltpu.make_async_copy(
                zero_v.at[pl.ds(0, 8)], deg_sh.at[pl.ds(sid * 8, 8)],
                semz).wait()

        plsc.subcore_barrier()

        def step(i, cur_idx, cur_dst, cur_rows, semg_c, sems_c, semi_c,
                 nxt_idx, nxt_rows, semg_n, sems_n, semi_n):
            @pl.when(i + 1 < myc)
            def _start_next():
                @pl.when(i >= 1)
                def _reuse():
                    wait_dma(nxt_idx, nxt_rows, sems_n)
                wait_idx(e_hbm, nxt_idx, semi_n)
                start_gather(nxt_idx, nxt_rows, semg_n)

            wait_dma(cur_idx, cur_rows, semg_c)
            for g in range(CHUNK // L):
                cur_dst[pl.ds(g * L, L)] = cur_idx[1, pl.ds(g * L, L)]

            @pl.when(i + 2 < myc)
            def _prefetch_idx():
                start_idx_load(e_hbm, chunk_off(i + 2), cur_idx, semi_c)

            pltpu.async_copy(cur_rows, acc_sh.at[cur_dst], sems_c, add=True)
            do_degree(cur_dst)

        @pl.loop(0, CT + 1)
        def _chunk(i):
            @pl.when(i < myc)
            def _active():
                @pl.when(lax.bitwise_and(i, 1) == 0)
                def _even():
                    step(i, idxa_v, dsta_v, rowsa_v, semga, semsa, semia,
                         idxb_v, rowsb_v, semgb, semsb, semib)

                @pl.when(lax.bitwise_and(i, 1) == 1)
                def _odd():
                    step(i, idxb_v, dstb_v, rowsb_v, semgb, semsb, semib,
                         idxa_v, rowsa_v, semga, semsa, semia)

        wait_dma(idxa_v, rowsa_v, semsa)

        @pl.when(myc >= 2)
        def _drain_b():
            wait_dma(idxb_v, rowsb_v, semsb)

        plsc.subcore_barrier()

        pltpu.sync_copy(degp_v, deg_sh.at[idxdr_v], add=True)

        plsc.subcore_barrier()

        pltpu.sync_copy(acc_sh.at[pl.ds(my_rows, RT)],
                        out_hbm.at[r, pl.ds(my_rows, RT)])

        @pl.when(sid < DR // 8)
        def _write_deg():
            pltpu.sync_copy(deg_sh.at[pl.ds(sid * 8, 8)],
                            deg_hbm.at[r, pl.ds(sid * 8, 8)])

    @pl.when(cid == 0)
    def _half0():
        do_relation(e0_hbm, 0)
        do_relation(e1_hbm, 1)

    @pl.when(cid == 1)
    def _half1():
        do_relation(e2_hbm, 2)
        do_relation(e3_hbm, 3)


BL = 1024
SB = BL // D


def _tc_body(coeffs_ref, acc_ref, deg_ref, bases_ref, bias_ref, out_ref):
    for s in range(SB):
        y0 = jnp.zeros((D, DOUT), jnp.float32)
        y1 = jnp.zeros((D, DOUT), jnp.float32)
        for r in range(R):
            rec = 1.0 / jnp.maximum(deg_ref[r], 1.0)
            rec_t = jnp.transpose(rec)
            nrm = acc_ref[r, s * D:(s + 1) * D, :] * rec_t[:, s:s + 1]
            y0 = y0 + coeffs_ref[r, 0] * nrm
            y1 = y1 + coeffs_ref[r, 1] * nrm
        h = jnp.dot(y0, bases_ref[0], preferred_element_type=jnp.float32)
        h = h + jnp.dot(y1, bases_ref[1], preferred_element_type=jnp.float32)
        out_ref[s * D:(s + 1) * D, :] = h + bias_ref[...]


_tc_combine = pl.pallas_call(
    _tc_body,
    grid=(NPAD // BL,),
    in_specs=[
        pl.BlockSpec(memory_space=pltpu.SMEM),
        pl.BlockSpec((R, BL, D), lambda i: (0, i, 0)),
        pl.BlockSpec((R, SB, D), lambda i: (0, i, 0)),
        pl.BlockSpec((NB, DIN, DOUT), lambda i: (0, 0, 0)),
        pl.BlockSpec((1, DOUT), lambda i: (0, 0)),
    ],
    out_specs=pl.BlockSpec((BL, DOUT), lambda i: (i, 0)),
    out_shape=jax.ShapeDtypeStruct((N, DOUT), jnp.float32),
)


def kernel(x, edge_index_r0, edge_index_r1, edge_index_r2, edge_index_r3,
           basis_coeffs, bases, h_bias):
    acc, deg = _sc_aggregate(x, edge_index_r0, edge_index_r1, edge_index_r2,
                             edge_index_r3)
    return _tc_combine(basis_coeffs, acc, deg, bases, h_bias.reshape(1, DOUT))


# speedup vs baseline: 1.0013x; 1.0013x over previous
import functools

import jax
import jax.numpy as jnp
from jax import lax
from jax.experimental import pallas as pl
from jax.experimental.pallas import tpu as pltpu
from jax.experimental.pallas import tpu_sc as plsc

N = 10000
E = 80000
DIN = 128
DOUT = 128
R = 4
NB = 2

NC = 2
NS = 16
L = 16

D = 128
NPAD = 10240
RT = NPAD // NS
DR = NPAD // D
CHUNK = 128
NCH = E // CHUNK
CT = NCH // NS
ET = CT * CHUNK
ZR = 16

_mesh = plsc.VectorSubcoreMesh(
    core_axis_name="c", subcore_axis_name="s", num_cores=NC, num_subcores=NS
)


@functools.partial(
    pl.kernel,
    out_type=(
        jax.ShapeDtypeStruct((R, NPAD, D), jnp.float32),
        jax.ShapeDtypeStruct((R, DR, D), jnp.float32),
    ),
    mesh=_mesh,
    compiler_params=pltpu.CompilerParams(needs_layout_passes=False),
    scratch_types=[
        pltpu.VMEM((2, CHUNK), jnp.int32),
        pltpu.VMEM((2, CHUNK), jnp.int32),
        pltpu.VMEM((CHUNK,), jnp.int32),
        pltpu.VMEM((CHUNK,), jnp.int32),
        pltpu.VMEM((CHUNK, D), jnp.float32),
        pltpu.VMEM((CHUNK, D), jnp.float32),
        pltpu.VMEM((ZR, D), jnp.float32),
        pltpu.VMEM((DR, D), jnp.float32),
        pltpu.VMEM((DR,), jnp.int32),
        pltpu.VMEM_SHARED((NPAD, D), jnp.float32),
        pltpu.VMEM_SHARED((DR, D), jnp.float32),
        pltpu.SemaphoreType.DMA,
        pltpu.SemaphoreType.DMA,
        pltpu.SemaphoreType.DMA,
        pltpu.SemaphoreType.DMA,
        pltpu.SemaphoreType.DMA,
        pltpu.SemaphoreType.DMA,
        pltpu.SemaphoreType.DMA,
    ],
)
def _sc_aggregate(x_hbm, e0_hbm, e1_hbm, e2_hbm, e3_hbm, out_hbm, deg_hbm,
                  idxa_v, idxb_v, dsta_v, dstb_v, rowsa_v, rowsb_v, zero_v,
                  degp_v, idxdr_v, acc_sh, deg_sh, semga, semgb, semsa, semsb,
                  semia, semib, semz):
    cid = lax.axis_index("c")
    sid = lax.axis_index("s")

    @pl.loop(0, ZR * D // L)
    def _fill(i):
        zero_v[i // (D // L), pl.ds((i % (D // L)) * L, L)] = (
            jnp.zeros((L,), jnp.float32))

    @pl.loop(0, DR // L)
    def _fill_idx(g):
        idxdr_v[pl.ds(g * L, L)] = lax.iota(jnp.int32, L) + g * L

    my_rows = sid * RT
    base_e = sid * ET
    myc = CT + jnp.where(sid == 0, 1, 0)

    def chunk_off(c):
        return jnp.where(c < CT, base_e + c * CHUNK, (NCH - 1) * CHUNK)

    def start_idx_load(e_hbm, off, idx_v, semi):
        pltpu.async_copy(e_hbm.at[:, pl.ds(off, CHUNK)], idx_v, semi)

    def wait_idx(e_hbm, idx_v, semi):
        pltpu.make_async_copy(e_hbm.at[:, pl.ds(0, CHUNK)], idx_v,
                              semi).wait()

    def wait_dma(idx_v, rows_v, sem):
        pltpu.make_async_copy(x_hbm.at[idx_v.at[0]], rows_v, sem).wait()

    def do_degree(dst_v):
        lane = lax.iota(jnp.int32, L)
        one = jnp.ones((L,), jnp.float32)
        for g in range(CHUNK // L):
            dst = dst_v[pl.ds(g * L, L)]
            hi = lax.shift_right_logical(dst, 7)
            lo = lax.bitwise_and(dst, 127)
            for k in range(L):
                plsc.addupdate_scatter(degp_v, [hi, lo], one,
                                       mask=lane == k)

    def do_relation(e_hbm, r):
        start_idx_load(e_hbm, chunk_off(0), idxa_v, semia)

        @pl.when(myc >= 2)
        def _prime_idx1():
            start_idx_load(e_hbm, chunk_off(1), idxb_v, semib)
        wait_idx(e_hbm, idxa_v, semia)
        pltpu.async_copy(x_hbm.at[idxa_v.at[0]], rowsa_v, semga)

        @pl.loop(0, RT // ZR)
        def _clear(z):
            pltpu.async_copy(zero_v, acc_sh.at[pl.ds(my_rows + z * ZR, ZR)],
                             semz)

        @pl.when(sid < DR // 8)
        def _clear_deg():
            pltpu.async_copy(zero_v.at[pl.ds(0, 8)],
                             deg_sh.at[pl.ds(sid * 8, 8)], semz)

        @pl.loop(0, DR * (D // L))
        def _clear_degp(i):
            degp_v[i // (D // L), pl.ds((i % (D // L)) * L, L)] = (
                jnp.zeros((L,), jnp.float32))

        @pl.loop(0, RT // ZR)
        def _drain_clear(z):
            pltpu.make_async_copy(
                zero_v, acc_sh.at[pl.ds(my_rows, ZR)], semz).wait()

        @pl.when(sid < DR // 8)
        def _drain_clear_deg():
            pltpu.make_async_copy(
                zero_v.at[pl.ds(0, 8)], deg_sh.at[pl.ds(sid * 8, 8)],
                semz).wait()

        plsc.subcore_barrier()

        def step(i, cur_idx, cur_dst, cur_rows, semg_c, sems_c, semi_c,
                 nxt_idx, nxt_rows, semg_n, sems_n, semi_n):
            @pl.when(i + 1 < myc)
            def _start_next():
                @pl.when(i >= 1)
                def _reuse():
                    wait_dma(nxt_idx, nxt_rows, sems_n)
                wait_idx(e_hbm, nxt_idx, semi_n)
                pltpu.async_copy(x_hbm.at[nxt_idx.at[0]], nxt_rows, semg_n)

            wait_dma(cur_idx, cur_rows, semg_c)
            for g in range(CHUNK // L):
                cur_dst[pl.ds(g * L, L)] = cur_idx[1, pl.ds(g * L, L)]

            @pl.when(i + 2 < myc)
            def _prefetch_idx():
                start_idx_load(e_hbm, chunk_off(i + 2), cur_idx, semi_c)

            pltpu.async_copy(cur_rows, acc_sh.at[cur_dst], sems_c, add=True)
            do_degree(cur_dst)

        @pl.loop(0, CT + 1)
        def _chunk(i):
            @pl.when(i < myc)
            def _active():
                @pl.when(lax.bitwise_and(i, 1) == 0)
                def _even():
                    step(i, idxa_v, dsta_v, rowsa_v, semga, semsa, semia,
                         idxb_v, rowsb_v, semgb, semsb, semib)

                @pl.when(lax.bitwise_and(i, 1) == 1)
                def _odd():
                    step(i, idxb_v, dstb_v, rowsb_v, semgb, semsb, semib,
                         idxa_v, rowsa_v, semga, semsa, semia)

        wait_dma(idxa_v, rowsa_v, semsa)

        @pl.when(myc >= 2)
        def _drain_b():
            wait_dma(idxb_v, rowsb_v, semsb)

        plsc.subcore_barrier()

        pltpu.sync_copy(degp_v, deg_sh.at[idxdr_v], add=True)

        plsc.subcore_barrier()

        pltpu.sync_copy(acc_sh.at[pl.ds(my_rows, RT)],
                        out_hbm.at[r, pl.ds(my_rows, RT)])

        @pl.when(sid < DR // 8)
        def _write_deg():
            pltpu.sync_copy(deg_sh.at[pl.ds(sid * 8, 8)],
                            deg_hbm.at[r, pl.ds(sid * 8, 8)])

    @pl.when(cid == 0)
    def _half0():
        do_relation(e0_hbm, 0)
        do_relation(e1_hbm, 1)

    @pl.when(cid == 1)
    def _half1():
        do_relation(e2_hbm, 2)
        do_relation(e3_hbm, 3)


BL = 1024
SB = BL // D


def _tc_body(coeffs_ref, acc_ref, deg_ref, bases_ref, bias_ref, out_ref):
    for s in range(SB):
        y0 = jnp.zeros((D, DOUT), jnp.float32)
        y1 = jnp.zeros((D, DOUT), jnp.float32)
        for r in range(R):
            rec = 1.0 / jnp.maximum(deg_ref[r], 1.0)
            rec_t = jnp.transpose(rec)
            nrm = acc_ref[r, s * D:(s + 1) * D, :] * rec_t[:, s:s + 1]
            y0 = y0 + coeffs_ref[r, 0] * nrm
            y1 = y1 + coeffs_ref[r, 1] * nrm
        h = jnp.dot(y0, bases_ref[0], preferred_element_type=jnp.float32)
        h = h + jnp.dot(y1, bases_ref[1], preferred_element_type=jnp.float32)
        out_ref[s * D:(s + 1) * D, :] = h + bias_ref[...]


_tc_combine = pl.pallas_call(
    _tc_body,
    grid=(NPAD // BL,),
    in_specs=[
        pl.BlockSpec(memory_space=pltpu.SMEM),
        pl.BlockSpec((R, BL, D), lambda i: (0, i, 0)),
        pl.BlockSpec((R, SB, D), lambda i: (0, i, 0)),
        pl.BlockSpec((NB, DIN, DOUT), lambda i: (0, 0, 0)),
        pl.BlockSpec((1, DOUT), lambda i: (0, 0)),
    ],
    out_specs=pl.BlockSpec((BL, DOUT), lambda i: (i, 0)),
    out_shape=jax.ShapeDtypeStruct((N, DOUT), jnp.float32),
)


def kernel(x, edge_index_r0, edge_index_r1, edge_index_r2, edge_index_r3,
           basis_coeffs, bases, h_bias):
    acc, deg = _sc_aggregate(x, edge_index_r0, edge_index_r1, edge_index_r2,
                             edge_index_r3)
    return _tc_combine(basis_coeffs, acc, deg, bases, h_bias.reshape(1, DOUT))


# speedup vs baseline: 1.0034x; 1.0020x over previous
import functools

import jax
import jax.numpy as jnp
from jax import lax
from jax.experimental import pallas as pl
from jax.experimental.pallas import tpu as pltpu
from jax.experimental.pallas import tpu_sc as plsc

N = 10000
E = 80000
DIN = 128
DOUT = 128
R = 4
NB = 2

NC = 2
NS = 16
L = 16

D = 128
NPAD = 10240
RT = NPAD // NS
DR = NPAD // D
CHUNK = 128
NCH = E // CHUNK
CT = NCH // NS
ET = CT * CHUNK
ZR = 16

_mesh = plsc.VectorSubcoreMesh(
    core_axis_name="c", subcore_axis_name="s", num_cores=NC, num_subcores=NS
)


@functools.partial(
    pl.kernel,
    out_type=(
        jax.ShapeDtypeStruct((R, NPAD, D), jnp.float32),
        jax.ShapeDtypeStruct((R, DR, D), jnp.float32),
    ),
    mesh=_mesh,
    compiler_params=pltpu.CompilerParams(needs_layout_passes=False),
    scratch_types=[
        pltpu.VMEM((2, CHUNK), jnp.int32),
        pltpu.VMEM((2, CHUNK), jnp.int32),
        pltpu.VMEM((CHUNK,), jnp.int32),
        pltpu.VMEM((CHUNK,), jnp.int32),
        pltpu.VMEM((CHUNK, D), jnp.float32),
        pltpu.VMEM((CHUNK, D), jnp.float32),
        pltpu.VMEM((ZR, D), jnp.float32),
        pltpu.VMEM((DR, D), jnp.float32),
        pltpu.VMEM((DR,), jnp.int32),
        pltpu.VMEM_SHARED((NPAD, D), jnp.float32),
        pltpu.VMEM_SHARED((DR, D), jnp.float32),
        pltpu.SemaphoreType.DMA,
        pltpu.SemaphoreType.DMA,
        pltpu.SemaphoreType.DMA,
        pltpu.SemaphoreType.DMA,
        pltpu.SemaphoreType.DMA,
        pltpu.SemaphoreType.DMA,
        pltpu.SemaphoreType.DMA,
    ],
)
def _sc_aggregate(x_hbm, e0_hbm, e1_hbm, e2_hbm, e3_hbm, out_hbm, deg_hbm,
                  idxa_v, idxb_v, dsta_v, dstb_v, rowsa_v, rowsb_v, zero_v,
                  degp_v, idxdr_v, acc_sh, deg_sh, semga, semgb, semsa, semsb,
                  semia, semib, semz):
    cid = lax.axis_index("c")
    sid = lax.axis_index("s")

    @pl.loop(0, ZR * D // L)
    def _fill(i):
        zero_v[i // (D // L), pl.ds((i % (D // L)) * L, L)] = (
            jnp.zeros((L,), jnp.float32))

    @pl.loop(0, DR // L)
    def _fill_idx(g):
        idxdr_v[pl.ds(g * L, L)] = lax.iota(jnp.int32, L) + g * L

    my_rows = sid * RT
    base_e = sid * ET
    myc = CT + jnp.where(sid == 0, 1, 0)

    def chunk_off(c):
        return jnp.where(c < CT, base_e + c * CHUNK, (NCH - 1) * CHUNK)

    def start_idx_load(e_hbm, off, idx_v, semi):
        pltpu.async_copy(e_hbm.at[:, pl.ds(off, CHUNK)], idx_v, semi)

    def wait_idx(e_hbm, idx_v, semi):
        pltpu.make_async_copy(e_hbm.at[:, pl.ds(0, CHUNK)], idx_v,
                              semi).wait()

    def wait_dma(idx_v, rows_v, sem):
        pltpu.make_async_copy(x_hbm.at[idx_v.at[0]], rows_v, sem).wait()

    def do_degree(dst_v):
        lane = lax.iota(jnp.int32, L)
        one = jnp.ones((L,), jnp.float32)
        for g in range(CHUNK // L):
            dst = dst_v[pl.ds(g * L, L)]
            hi = lax.shift_right_logical(dst, 7)
            lo = lax.bitwise_and(dst, 127)
            for k in range(L):
                plsc.addupdate_scatter(degp_v, [hi, lo], one,
                                       mask=lane == k)

    def do_relation(e_hbm, r):
        start_idx_load(e_hbm, chunk_off(0), idxa_v, semia)

        @pl.when(myc >= 2)
        def _prime_idx1():
            start_idx_load(e_hbm, chunk_off(1), idxb_v, semib)
        wait_idx(e_hbm, idxa_v, semia)
        pltpu.async_copy(x_hbm.at[idxa_v.at[0]], rowsa_v, semga)

        @pl.loop(0, RT // ZR)
        def _clear(z):
            pltpu.async_copy(zero_v, acc_sh.at[pl.ds(my_rows + z * ZR, ZR)],
                             semz)

        @pl.when(sid < DR // 8)
        def _clear_deg():
            pltpu.async_copy(zero_v.at[pl.ds(0, 8)],
                             deg_sh.at[pl.ds(sid * 8, 8)], semz)

        @pl.loop(0, DR * (D // L))
        def _clear_degp(i):
            degp_v[i // (D // L), pl.ds((i % (D // L)) * L, L)] = (
                jnp.zeros((L,), jnp.float32))

        @pl.loop(0, RT // ZR)
        def _drain_clear(z):
            pltpu.make_async_copy(
                zero_v, acc_sh.at[pl.ds(my_rows, ZR)], semz).wait()

        @pl.when(sid < DR // 8)
        def _drain_clear_deg():
            pltpu.make_async_copy(
                zero_v.at[pl.ds(0, 8)], deg_sh.at[pl.ds(sid * 8, 8)],
                semz).wait()

        plsc.subcore_barrier()

        def step(i, cur_idx, cur_dst, cur_rows, semg_c, sems_c, semi_c,
                 nxt_idx, nxt_rows, semg_n, sems_n, semi_n):
            @pl.when(i + 1 < myc)
            def _start_next():
                @pl.when(i >= 1)
                def _reuse():
                    wait_dma(nxt_idx, nxt_rows, sems_n)
                wait_idx(e_hbm, nxt_idx, semi_n)
                pltpu.async_copy(x_hbm.at[nxt_idx.at[0]], nxt_rows, semg_n)

            for g in range(CHUNK // L):
                cur_dst[pl.ds(g * L, L)] = cur_idx[1, pl.ds(g * L, L)]
            do_degree(cur_dst)
            wait_dma(cur_idx, cur_rows, semg_c)
            pltpu.async_copy(cur_rows, acc_sh.at[cur_dst], sems_c, add=True)

            @pl.when(i + 2 < myc)
            def _prefetch_idx():
                start_idx_load(e_hbm, chunk_off(i + 2), cur_idx, semi_c)

        @pl.loop(0, CT + 1)
        def _chunk(i):
            @pl.when(i < myc)
            def _active():
                @pl.when(lax.bitwise_and(i, 1) == 0)
                def _even():
                    step(i, idxa_v, dsta_v, rowsa_v, semga, semsa, semia,
                         idxb_v, rowsb_v, semgb, semsb, semib)

                @pl.when(lax.bitwise_and(i, 1) == 1)
                def _odd():
                    step(i, idxb_v, dstb_v, rowsb_v, semgb, semsb, semib,
                         idxa_v, rowsa_v, semga, semsa, semia)

        wait_dma(idxa_v, rowsa_v, semsa)

        @pl.when(myc >= 2)
        def _drain_b():
            wait_dma(idxb_v, rowsb_v, semsb)

        plsc.subcore_barrier()

        pltpu.sync_copy(degp_v, deg_sh.at[idxdr_v], add=True)

        plsc.subcore_barrier()

        pltpu.sync_copy(acc_sh.at[pl.ds(my_rows, RT)],
                        out_hbm.at[r, pl.ds(my_rows, RT)])

        @pl.when(sid < DR // 8)
        def _write_deg():
            pltpu.sync_copy(deg_sh.at[pl.ds(sid * 8, 8)],
                            deg_hbm.at[r, pl.ds(sid * 8, 8)])

    @pl.when(cid == 0)
    def _half0():
        do_relation(e0_hbm, 0)
        do_relation(e1_hbm, 1)

    @pl.when(cid == 1)
    def _half1():
        do_relation(e2_hbm, 2)
        do_relation(e3_hbm, 3)


BL = 1024
SB = BL // D


def _tc_body(coeffs_ref, acc_ref, deg_ref, bases_ref, bias_ref, out_ref):
    for s in range(SB):
        y0 = jnp.zeros((D, DOUT), jnp.float32)
        y1 = jnp.zeros((D, DOUT), jnp.float32)
        for r in range(R):
            rec = 1.0 / jnp.maximum(deg_ref[r], 1.0)
            rec_t = jnp.transpose(rec)
            nrm = acc_ref[r, s * D:(s + 1) * D, :] * rec_t[:, s:s + 1]
            y0 = y0 + coeffs_ref[r, 0] * nrm
            y1 = y1 + coeffs_ref[r, 1] * nrm
        h = jnp.dot(y0, bases_ref[0], preferred_element_type=jnp.float32)
        h = h + jnp.dot(y1, bases_ref[1], preferred_element_type=jnp.float32)
        out_ref[s * D:(s + 1) * D, :] = h + bias_ref[...]


_tc_combine = pl.pallas_call(
    _tc_body,
    grid=(NPAD // BL,),
    in_specs=[
        pl.BlockSpec(memory_space=pltpu.SMEM),
        pl.BlockSpec((R, BL, D), lambda i: (0, i, 0)),
        pl.BlockSpec((R, SB, D), lambda i: (0, i, 0)),
        pl.BlockSpec((NB, DIN, DOUT), lambda i: (0, 0, 0)),
        pl.BlockSpec((1, DOUT), lambda i: (0, 0)),
    ],
    out_specs=pl.BlockSpec((BL, DOUT), lambda i: (i, 0)),
    out_shape=jax.ShapeDtypeStruct((N, DOUT), jnp.float32),
)


def kernel(x, edge_index_r0, edge_index_r1, edge_index_r2, edge_index_r3,
           basis_coeffs, bases, h_bias):
    acc, deg = _sc_aggregate(x, edge_index_r0, edge_index_r1, edge_index_r2,
                             edge_index_r3)
    return _tc_combine(basis_coeffs, acc, deg, bases, h_bias.reshape(1, DOUT))
